# baseline (speedup 1.0000x reference)
import functools

import jax
import jax.numpy as jnp
import numpy as np
from jax import lax
from jax.experimental import pallas as pl
from jax.experimental.pallas import tpu as pltpu
from jax.experimental.pallas import tpu_sc as plsc

_NC = 2
_NS = 16
_NW = _NC * _NS
_CH = 128
_K = 5
_TW = 2048


def _eye(dim):
    return (lax.broadcasted_iota(jnp.int32, (dim, dim), 0)
            == lax.broadcasted_iota(jnp.int32, (dim, dim), 1)
            ).astype(jnp.float32)


def _tc_table_rows(table_t, vocab, dim):
    nblk = -(-vocab // _TW)
    q = _TW // 4

    def body(in_ref, out_ref):
        eye = _eye(dim)
        for k in range(4):
            out_ref[:, k * dim:(k + 1) * dim] = lax.dot_general(
                in_ref[:, k * q:(k + 1) * q], eye, (((0,), (0,)), ((), ())),
                precision=lax.Precision.HIGHEST)

    return pl.pallas_call(
        body,
        grid=(nblk,),
        in_specs=[pl.BlockSpec((dim, _TW), lambda i: (0, i))],
        out_specs=pl.BlockSpec((q, 128), lambda i: (i, 0)),
        out_shape=jax.ShapeDtypeStruct((nblk * q, 128), table_t.dtype),
        compiler_params=pltpu.CompilerParams(
            dimension_semantics=("parallel",)),
    )(table_t)


def _sc_gather_scatter(staging, idx, oidx, n, dim):
    per_w = n // _NW
    nch = per_w // _CH
    nsuper = nch // _K
    assert per_w * _NW == n and nsuper * _K * _CH == per_w and nsuper % 2 == 0
    mesh = plsc.VectorSubcoreMesh(core_axis_name="c", subcore_axis_name="s")

    @functools.partial(
        pl.kernel,
        mesh=mesh,
        out_type=jax.ShapeDtypeStruct((n, dim), staging.dtype),
        compiler_params=pltpu.CompilerParams(use_tc_tiling_on_sc=False),
        scratch_types=[
            pltpu.VMEM((nch, _CH), jnp.int32),
            pltpu.VMEM((nch, _CH), jnp.int32),
            pltpu.VMEM((2, _K, _CH, dim), jnp.float32),
            pltpu.SemaphoreType.DMA,
            pltpu.SemaphoreType.DMA,
            pltpu.SemaphoreType.DMA,
            pltpu.SemaphoreType.DMA,
        ],
    )
    def gather_kernel(tab_hbm, idx_hbm, oidx_hbm, out_hbm, idx_v, oidx_v,
                      rows_v, gsem0, gsem1, osem0, osem1):
        wid = lax.axis_index("s") * _NC + lax.axis_index("c")
        pltpu.sync_copy(idx_hbm.at[pl.ds(wid * nch, nch)], idx_v)
        pltpu.sync_copy(oidx_hbm.at[pl.ds(wid * nch, nch)], oidx_v)
        gsems = (gsem0, gsem1)
        osems = (osem0, osem1)

        def fire_gathers(s, b):
            for j in range(_K):
                c = s * _K + j
                pltpu.async_copy(tab_hbm.at[idx_v.at[c]],
                                 rows_v.at[b].at[j], gsems[b])

        def drain_gathers(s, b):
            for j in range(_K):
                c = s * _K + j
                pltpu.make_async_copy(tab_hbm.at[idx_v.at[c]],
                                      rows_v.at[b].at[j], gsems[b]).wait()

        def fire_stores(s, b):
            for j in range(_K):
                c = s * _K + j
                pltpu.async_copy(rows_v.at[b].at[j],
                                 out_hbm.at[oidx_v.at[c]], osems[b])

        def drain_stores(s, b):
            for j in range(_K):
                c = s * _K + j
                pltpu.make_async_copy(rows_v.at[b].at[j],
                                      out_hbm.at[oidx_v.at[c]],
                                      osems[b]).wait()

        for s in range(2):
            fire_gathers(s, s)
            drain_gathers(s, s)
            fire_stores(s, s)

        @pl.loop(2, nsuper, step=2)
        def _(s0):
            for b in range(2):
                s = s0 + b
                drain_stores(s - 2, b)
                fire_gathers(s, b)
                drain_gathers(s, b)
                fire_stores(s, b)

        drain_stores(nsuper - 2, 0)
        drain_stores(nsuper - 1, 1)

    idx2 = idx.reshape(n // _CH, _CH)
    oidx2 = oidx.reshape(n // _CH, _CH)
    return gather_kernel(staging, idx2, oidx2)


def _tc_retile(rows, hist, dim, batch):
    rows128 = rows.reshape(hist * batch * dim // 128, 128)
    blk = batch // 4
    q = batch // 4

    def body(in_ref, out_ref):
        eye = _eye(dim)
        for k in range(4):
            out_ref[0, :, k * q:(k + 1) * q] = lax.dot_general(
                eye, in_ref[:, k * dim:(k + 1) * dim],
                (((1,), (1,)), ((), ())), precision=lax.Precision.HIGHEST)

    return pl.pallas_call(
        body,
        grid=(hist,),
        in_specs=[pl.BlockSpec((blk, 128), lambda h: (h, 0))],
        out_specs=pl.BlockSpec((1, dim, batch), lambda h: (h, 0, 0)),
        out_shape=jax.ShapeDtypeStruct((hist, dim, batch), rows.dtype),
        compiler_params=pltpu.CompilerParams(
            dimension_semantics=("parallel",)),
    )(rows128)


def kernel(x, table):
    batch, hist = x.shape
    vocab, dim = table.shape
    n = batch * hist

    table_t = jnp.swapaxes(table, 0, 1)
    staging = _tc_table_rows(table_t, vocab, dim)
    srows = staging.shape[0] * (128 // dim)
    staging = staging.reshape(srows, dim)

    v = x.reshape(n).astype(jnp.int32)
    c = v % _TW
    idx = (v - c) + 4 * (c % (_TW // 4)) + c // (_TW // 4)

    i = np.arange(n, dtype=np.int64)
    b, h = i // hist, i % hist
    qb = batch // 4
    oidx = jnp.asarray(h * batch + 4 * (b % qb) + b // qb, dtype=jnp.int32)

    rows = _sc_gather_scatter(staging, idx, oidx, n, dim)
    out_t = _tc_retile(rows, hist, dim, batch)
    return out_t.transpose(2, 0, 1)


# speedup vs baseline: 2.7854x; 2.7854x over previous
import functools

import jax
import jax.numpy as jnp
import numpy as np
from jax import lax
from jax.experimental import pallas as pl
from jax.experimental.pallas import tpu as pltpu
from jax.experimental.pallas import tpu_sc as plsc

_NC = 2
_NS = 16
_NW = _NC * _NS
_CH = 128
_K = 5
_TW = 2048


def _eye(dim):
    return (lax.broadcasted_iota(jnp.int32, (dim, dim), 0)
            == lax.broadcasted_iota(jnp.int32, (dim, dim), 1)
            ).astype(jnp.float32)


def _tc_table_rows(table_t, vocab, dim):
    w = _TW * 4
    nblk = -(-vocab // w)

    def body(in_ref, out_ref):
        s = jnp.concatenate(
            [in_ref[:, a * _TW:(a + 1) * _TW] for a in range(4)], axis=0)
        out_ref[...] = s.T

    return pl.pallas_call(
        body,
        grid=(nblk,),
        in_specs=[pl.BlockSpec((dim, w), lambda i: (0, i))],
        out_specs=pl.BlockSpec((_TW, 128), lambda i: (i, 0)),
        out_shape=jax.ShapeDtypeStruct((nblk * _TW, 128), table_t.dtype),
        compiler_params=pltpu.CompilerParams(
            dimension_semantics=("parallel",)),
    )(table_t)


def _sc_gather_scatter(staging, idx, oidx, n, dim):
    per_w = n // _NW
    nch = per_w // _CH
    nsuper = nch // _K
    assert per_w * _NW == n and nsuper * _K * _CH == per_w and nsuper % 2 == 0
    mesh = plsc.VectorSubcoreMesh(core_axis_name="c", subcore_axis_name="s")

    @functools.partial(
        pl.kernel,
        mesh=mesh,
        out_type=jax.ShapeDtypeStruct((n, dim), staging.dtype),
        compiler_params=pltpu.CompilerParams(use_tc_tiling_on_sc=False),
        scratch_types=[
            pltpu.VMEM((nch, _CH), jnp.int32),
            pltpu.VMEM((nch, _CH), jnp.int32),
            pltpu.VMEM((2, _K, _CH, dim), jnp.float32),
            pltpu.SemaphoreType.DMA,
            pltpu.SemaphoreType.DMA,
            pltpu.SemaphoreType.DMA,
            pltpu.SemaphoreType.DMA,
        ],
    )
    def gather_kernel(tab_hbm, idx_hbm, oidx_hbm, out_hbm, idx_v, oidx_v,
                      rows_v, gsem0, gsem1, osem0, osem1):
        wid = lax.axis_index("s") * _NC + lax.axis_index("c")
        pltpu.sync_copy(idx_hbm.at[pl.ds(wid * nch, nch)], idx_v)
        pltpu.sync_copy(oidx_hbm.at[pl.ds(wid * nch, nch)], oidx_v)
        gsems = (gsem0, gsem1)
        osems = (osem0, osem1)

        def fire_gathers(s, b):
            for j in range(_K):
                c = s * _K + j
                pltpu.async_copy(tab_hbm.at[idx_v.at[c]],
                                 rows_v.at[b].at[j], gsems[b])

        def drain_gathers(s, b):
            for j in range(_K):
                c = s * _K + j
                pltpu.make_async_copy(tab_hbm.at[idx_v.at[c]],
                                      rows_v.at[b].at[j], gsems[b]).wait()

        def fire_stores(s, b):
            for j in range(_K):
                c = s * _K + j
                pltpu.async_copy(rows_v.at[b].at[j],
                                 out_hbm.at[oidx_v.at[c]], osems[b])

        def drain_stores(s, b):
            for j in range(_K):
                c = s * _K + j
                pltpu.make_async_copy(rows_v.at[b].at[j],
                                      out_hbm.at[oidx_v.at[c]],
                                      osems[b]).wait()

        for s in range(2):
            fire_gathers(s, s)
            drain_gathers(s, s)
            fire_stores(s, s)

        @pl.loop(2, nsuper, step=2)
        def _(s0):
            for b in range(2):
                s = s0 + b
                drain_stores(s - 2, b)
                fire_gathers(s, b)
                drain_gathers(s, b)
                fire_stores(s, b)

        drain_stores(nsuper - 2, 0)
        drain_stores(nsuper - 1, 1)

    idx2 = idx.reshape(n // _CH, _CH)
    oidx2 = oidx.reshape(n // _CH, _CH)
    return gather_kernel(staging, idx2, oidx2)


def _tc_retile(rows, hist, dim, batch):
    rows128 = rows.reshape(hist * batch * dim // 128, 128)
    blk = batch // 4
    q = batch // 4

    def body(in_ref, out_ref):
        t = in_ref[...].T
        for k in range(4):
            out_ref[0, :, k * q:(k + 1) * q] = t[k * dim:(k + 1) * dim, :]

    return pl.pallas_call(
        body,
        grid=(hist,),
        in_specs=[pl.BlockSpec((blk, 128), lambda h: (h, 0))],
        out_specs=pl.BlockSpec((1, dim, batch), lambda h: (h, 0, 0)),
        out_shape=jax.ShapeDtypeStruct((hist, dim, batch), rows.dtype),
        compiler_params=pltpu.CompilerParams(
            dimension_semantics=("parallel",)),
    )(rows128)


def kernel(x, table):
    batch, hist = x.shape
    vocab, dim = table.shape
    n = batch * hist

    table_t = jnp.swapaxes(table, 0, 1)
    staging = _tc_table_rows(table_t, vocab, dim)
    srows = staging.shape[0] * (128 // dim)
    staging = staging.reshape(srows, dim)

    v = x.reshape(n).astype(jnp.int32)
    r = v % (_TW * 4)
    idx = (v - r) + 4 * (r % _TW) + r // _TW

    i = np.arange(n, dtype=np.int64)
    b, h = i // hist, i % hist
    qb = batch // 4
    oidx = jnp.asarray(h * batch + 4 * (b % qb) + b // qb, dtype=jnp.int32)

    rows = _sc_gather_scatter(staging, idx, oidx, n, dim)
    out_t = _tc_retile(rows, hist, dim, batch)
    return out_t.transpose(2, 0, 1)
